# baseline (speedup 1.0000x reference)
import functools

import jax
import jax.numpy as jnp
from jax import lax
from jax.experimental import pallas as pl
from jax.experimental.pallas import tpu as pltpu
from jax.experimental.pallas import tpu_sc as plsc

B = 4096
NEG = 16
D = 64
MARGIN = 24.0

NC = 2
NS = 16
L = 16
NW = NC * NS
PP = B // NW
PN = (B * NEG) // NW
C = 128
NCH = PN // C


def _dist_groups(hrows, rrows, trows, out_v, out_base, n_rows):
    def group(g, _):
        rowv = g * L + lax.broadcasted_iota(jnp.int32, (L,), 0)

        def dim(dd, acc):
            dsplat = jnp.zeros((L,), jnp.int32) + dd
            hv = plsc.load_gather(hrows, [rowv, dsplat])
            rv = plsc.load_gather(rrows, [rowv, dsplat])
            tv = plsc.load_gather(trows, [rowv, dsplat])
            return acc + jnp.abs(hv + rv - tv)

        acc = lax.fori_loop(0, D, dim, jnp.zeros((L,), jnp.float32))
        out_v[pl.ds(out_base + g * L, L)] = acc
        return _

    lax.fori_loop(0, n_rows // L, group, 0)


def _sc_body(ent_hbm, rel_hbm, hp_hbm, rp_hbm, tp_hbm, hn_hbm, rn_hbm, tn_hbm,
             w_hbm, posd_hbm, posdot_hbm, negd_hbm,
             idx_h, idx_r, idx_t, hrows, rrows, trows, wv,
             posd_v, posdot_v, negd_v, sem):
    wid = lax.axis_index("c") * NS + lax.axis_index("s")

    pltpu.sync_copy(w_hbm, wv)

    def load_rows(h_src, r_src, t_src, base):
        pltpu.sync_copy(h_src.at[pl.ds(base, C)], idx_h)
        pltpu.sync_copy(r_src.at[pl.ds(base, C)], idx_r)
        pltpu.sync_copy(t_src.at[pl.ds(base, C)], idx_t)
        c1 = pltpu.async_copy(ent_hbm.at[idx_h], hrows, sem)
        c2 = pltpu.async_copy(rel_hbm.at[idx_r], rrows, sem)
        c3 = pltpu.async_copy(ent_hbm.at[idx_t], trows, sem)
        c1.wait()
        c2.wait()
        c3.wait()

    load_rows(hp_hbm, rp_hbm, tp_hbm, wid * PP)
    wregs = [wv[pl.ds(k * L, L)] for k in range(D // L)]

    def pos_group(g, _):
        rowv = g * L + lax.broadcasted_iota(jnp.int32, (L,), 0)
        acc = jnp.zeros((L,), jnp.float32)
        dot = jnp.zeros((L,), jnp.float32)
        for d in range(D):
            dsplat = jnp.full((L,), d, jnp.int32)
            hv = plsc.load_gather(hrows, [rowv, dsplat])
            rv = plsc.load_gather(rrows, [rowv, dsplat])
            tv = plsc.load_gather(trows, [rowv, dsplat])
            s = hv + rv - tv
            acc = acc + jnp.abs(s)
            wd = jnp.take(wregs[d // L], jnp.full((L,), d % L, jnp.int32))
            dot = dot + s * wd
        posd_v[pl.ds(g * L, L)] = acc
        posdot_v[pl.ds(g * L, L)] = dot
        return _

    lax.fori_loop(0, PP // L, pos_group, 0)

    def neg_chunk(c, _):
        load_rows(hn_hbm, rn_hbm, tn_hbm, wid * PN + c * C)
        _dist_groups(hrows, rrows, trows, negd_v, c * C, C)
        return _

    lax.fori_loop(0, NCH, neg_chunk, 0)

    pltpu.sync_copy(posd_v, posd_hbm.at[pl.ds(wid * PP, PP)])
    pltpu.sync_copy(posdot_v, posdot_hbm.at[pl.ds(wid * PP, PP)])
    pltpu.sync_copy(negd_v, negd_hbm.at[pl.ds(wid * PN, PN)])


_sc_call = pl.kernel(
    _sc_body,
    out_type=[
        jax.ShapeDtypeStruct((B,), jnp.float32),
        jax.ShapeDtypeStruct((B,), jnp.float32),
        jax.ShapeDtypeStruct((B * NEG,), jnp.float32),
    ],
    mesh=plsc.VectorSubcoreMesh(core_axis_name="c", subcore_axis_name="s",
                                num_cores=NC, num_subcores=NS),
    compiler_params=pltpu.CompilerParams(needs_layout_passes=False,
                                         use_tc_tiling_on_sc=False),
    scratch_types=[
        pltpu.VMEM((C,), jnp.int32),
        pltpu.VMEM((C,), jnp.int32),
        pltpu.VMEM((C,), jnp.int32),
        pltpu.VMEM((C, D), jnp.float32),
        pltpu.VMEM((C, D), jnp.float32),
        pltpu.VMEM((C, D), jnp.float32),
        pltpu.VMEM((D,), jnp.float32),
        pltpu.VMEM((PP,), jnp.float32),
        pltpu.VMEM((PP,), jnp.float32),
        pltpu.VMEM((PN,), jnp.float32),
        pltpu.SemaphoreType.DMA,
    ],
)


def _combine_body(pd_ref, dot_ref, nd_ref, db_ref, o_ref):
    db = db_ref[0, 0]
    disc = jax.nn.sigmoid(dot_ref[...] + db)
    pos = -jnp.log(jax.nn.sigmoid(MARGIN - pd_ref[...]))
    neg = jnp.sum((1.0 / NEG) * jnp.log(jax.nn.sigmoid(MARGIN - nd_ref[...])),
                  axis=1, keepdims=True)
    o_ref[...] = disc * (pos + neg)


def _combine(pd, dot, nd, db):
    return pl.pallas_call(
        _combine_body,
        out_shape=jax.ShapeDtypeStruct((B, 1), jnp.float32),
    )(pd.reshape(B, 1), dot.reshape(B, 1), nd.reshape(B, NEG), db.reshape(1, 1))


def kernel(positive_triples, block_of_negative_triples, negative_sample_size,
           entities_emb, relations_emb, D_W, D_b):
    hp = positive_triples[:, 0]
    rp = positive_triples[:, 1]
    tp = positive_triples[:, 2]
    nflat = block_of_negative_triples.reshape(B * NEG, 3)
    hn = nflat[:, 0]
    rn = nflat[:, 1]
    tn = nflat[:, 2]
    w = D_W.reshape(D)

    posd, posdot, negd = _sc_call(entities_emb, relations_emb,
                                  hp, rp, tp, hn, rn, tn, w)
    out = _combine(posd, posdot, negd.reshape(B, NEG), D_b)
    return out.reshape(B)
